# Initial kernel scaffold; baseline (speedup 1.0000x reference)
import jax
import jax.numpy as jnp
from jax.experimental import pallas as pl


def kernel(x_node, x_edge, edge_index, W, b):
    raise NotImplementedError("write your pallas kernel here")


# speedup vs baseline: 4.2833x; 4.2833x over previous
import functools

import jax
import jax.numpy as jnp
from jax import lax
from jax.experimental import pallas as pl
from jax.experimental.pallas import tpu as pltpu
from jax.experimental.pallas import tpu_sc as plsc

_N_NODES = 10000
_N_EDGES = 320000
_D_FEAT = 128
_D_EDGE = 16

_NW = 32
_PER_W = _N_EDGES // _NW
_CE = 2000
_CHUNKS = _PER_W // _CE


def _tables_body(xn_ref, w0_ref, w1_ref, a_ref, b_ref):
    x = xn_ref[...]
    a_ref[...] = jnp.dot(x, w0_ref[...], preferred_element_type=jnp.float32)
    b_ref[...] = jnp.dot(x, w1_ref[...], preferred_element_type=jnp.float32)


def _edge_mm_body(xe_ref, w_ref, b_ref, o_ref):
    o_ref[...] = (
        jnp.dot(xe_ref[...], w_ref[...], preferred_element_type=jnp.float32)
        + b_ref[...]
    )


def _make_sc_combine():
    mesh = plsc.VectorSubcoreMesh(core_axis_name="c", subcore_axis_name="s")

    @functools.partial(
        pl.kernel,
        mesh=mesh,
        compiler_params=pltpu.CompilerParams(use_tc_tiling_on_sc=False),
        out_type=jax.ShapeDtypeStruct((_N_EDGES, _D_EDGE), jnp.float32),
        scratch_types=[
            pltpu.VMEM((_CE,), jnp.int32),
            pltpu.VMEM((_CE,), jnp.int32),
            pltpu.VMEM((_CE, _D_EDGE), jnp.float32),
            pltpu.VMEM((_CE, _D_EDGE), jnp.float32),
            pltpu.VMEM((_CE, _D_EDGE), jnp.float32),
            pltpu.SemaphoreType.DMA,
            pltpu.SemaphoreType.DMA,
        ],
    )
    def sc_combine(a_hbm, b_hbm, e0_hbm, e1_hbm, c_hbm, out_hbm,
                   idx0, idx1, ra, rb, acc, sem_a, sem_b):
        wid = lax.axis_index("s") * 2 + lax.axis_index("c")
        base = wid * _PER_W

        def chunk(j, carry):
            off = base + j * _CE
            pltpu.sync_copy(e0_hbm.at[pl.ds(off, _CE)], idx0)
            pltpu.sync_copy(e1_hbm.at[pl.ds(off, _CE)], idx1)
            cp_a = pltpu.async_copy(a_hbm.at[idx0], ra, sem_a)
            cp_b = pltpu.async_copy(b_hbm.at[idx1], rb, sem_b)
            pltpu.sync_copy(c_hbm.at[pl.ds(off, _CE)], acc)
            cp_a.wait()
            cp_b.wait()

            def row(i, c2):
                acc[i, :] = acc[i, :] + ra[i, :] + rb[i, :]
                return c2

            lax.fori_loop(0, _CE, row, 0)
            pltpu.sync_copy(acc, out_hbm.at[pl.ds(off, _CE)])
            return carry

        lax.fori_loop(0, _CHUNKS, chunk, 0)

    return sc_combine


_sc_combine = _make_sc_combine()


def kernel(x_node, x_edge, edge_index, W, b):
    e = edge_index.astype(jnp.int32)
    e0 = e[:, 0]
    e1 = e[:, 1]
    w0 = W[:_D_FEAT]
    w1 = W[_D_FEAT:2 * _D_FEAT]
    w2 = W[2 * _D_FEAT:]

    tab_a, tab_b = pl.pallas_call(
        _tables_body,
        out_shape=[
            jax.ShapeDtypeStruct((_N_NODES, _D_EDGE), jnp.float32),
            jax.ShapeDtypeStruct((_N_NODES, _D_EDGE), jnp.float32),
        ],
    )(x_node, w0, w1)

    w2_blk = jnp.kron(jnp.eye(8, dtype=jnp.float32), w2)
    b_blk = jnp.tile(b, 8)[None, :]
    xe2 = x_edge.reshape(_N_EDGES // 8, 8 * _D_EDGE)
    rows = _N_EDGES // 8
    blk = rows // 8
    c2 = pl.pallas_call(
        _edge_mm_body,
        grid=(8,),
        in_specs=[
            pl.BlockSpec((blk, 8 * _D_EDGE), lambda i: (i, 0)),
            pl.BlockSpec((8 * _D_EDGE, 8 * _D_EDGE), lambda i: (0, 0)),
            pl.BlockSpec((1, 8 * _D_EDGE), lambda i: (0, 0)),
        ],
        out_specs=pl.BlockSpec((blk, 8 * _D_EDGE), lambda i: (i, 0)),
        out_shape=jax.ShapeDtypeStruct((rows, 8 * _D_EDGE), jnp.float32),
    )(xe2, w2_blk, b_blk)
    c = c2.reshape(_N_EDGES, _D_EDGE)

    return _sc_combine(tab_a, tab_b, e0, e1, c)
